# baseline (speedup 1.0000x reference)
import functools

import jax
import jax.numpy as jnp
from jax import lax
from jax.experimental import pallas as pl
from jax.experimental.pallas import tpu as pltpu
from jax.experimental.pallas import tpu_sc as plsc

_D_EMBED = 64
_SCALE = float(_D_EMBED) ** 0.5
_LANES = 16
_NUM_WORKERS = 32
_NBUF = 2


def _make_lookup(nrow: int, ncol: int):
    assert nrow % (_NUM_WORKERS * _NBUF) == 0
    rpw = nrow // _NUM_WORKERS
    mesh = plsc.VectorSubcoreMesh(core_axis_name="c", subcore_axis_name="s")

    @functools.partial(
        pl.kernel,
        mesh=mesh,
        out_type=jax.ShapeDtypeStruct((nrow, ncol, _D_EMBED), jnp.float32),
        scratch_types=[
            pltpu.VMEM((rpw, ncol), jnp.int32),
            pltpu.VMEM((_NBUF, ncol, _D_EMBED), jnp.float32),
            pltpu.VMEM((_NBUF, ncol, _D_EMBED), jnp.float32),
            [pltpu.SemaphoreType.DMA] * _NBUF,
            [pltpu.SemaphoreType.DMA] * _NBUF,
        ],
        compiler_params=pltpu.CompilerParams(use_tc_tiling_on_sc=False),
    )
    def lookup(table_hbm, idx_hbm, out_hbm, idx_v, gbuf, wbuf, gsems, wsems):
        wid = lax.axis_index("s") * 2 + lax.axis_index("c")
        base = wid * rpw
        pltpu.sync_copy(idx_hbm.at[pl.ds(base, rpw)], idx_v)

        def gather_start(row, b):
            pltpu.async_copy(
                table_hbm.at[idx_v.at[row]], gbuf.at[b], gsems[b]
            )

        for b in range(_NBUF):
            gather_start(b, b)

        @pl.loop(0, rpw, step=_NBUF)
        def _(g0):
            for b in range(_NBUF):
                g = g0 + b

                @pl.when(g >= _NBUF)
                def _():
                    pltpu.make_async_copy(
                        wbuf.at[b], out_hbm.at[base], wsems[b]
                    ).wait()

                pltpu.make_async_copy(
                    table_hbm.at[idx_v.at[g]], gbuf.at[b], gsems[b]
                ).wait()

                @plsc.parallel_loop(0, ncol, unroll=8)
                def _(i):
                    for j in range(_D_EMBED // _LANES):
                        sl = pl.ds(j * _LANES, _LANES)
                        wbuf[b, i, sl] = gbuf[b, i, sl] * _SCALE

                pltpu.async_copy(wbuf.at[b], out_hbm.at[base + g], wsems[b])

                @pl.when(g + _NBUF < rpw)
                def _():
                    gather_start(g + _NBUF, b)

        for b in range(_NBUF):
            pltpu.make_async_copy(
                wbuf.at[b], out_hbm.at[base], wsems[b]
            ).wait()

    return lookup


def kernel(inp, emb_weight):
    b, t = inp.shape
    return _make_lookup(b, t)(emb_weight, inp)
